# Initial kernel scaffold; baseline (speedup 1.0000x reference)
import jax
import jax.numpy as jnp
from jax.experimental import pallas as pl


def kernel(x, edge_index, W1, b1, W2, b2, Wc, bc):
    raise NotImplementedError("write your pallas kernel here")


# speedup vs baseline: 41.0957x; 41.0957x over previous
import functools

import jax
import jax.numpy as jnp
from jax import lax
from jax.experimental import pallas as pl
from jax.experimental.pallas import tpu as pltpu
from jax.experimental.pallas import tpu_sc as plsc

_N = 100000
_E = 1600000
_F_IN = 34
_H = 8
_D2 = 2
_C = 4

_NC = 2
_NS = 16
_NW = _NC * _NS
_EPW = _E // _NW
_K = 5000
_NCHUNK = _EPW // _K
_NP = 100096
_TPR = _NP // _NS
_BT = 6256
_GT = _NP // _BT

_mesh = plsc.VectorSubcoreMesh(core_axis_name="c", subcore_axis_name="s")
_sc_params = pltpu.CompilerParams(use_tc_tiling_on_sc=False)


@functools.partial(
    pl.kernel,
    mesh=_mesh,
    compiler_params=_sc_params,
    out_type=(jax.ShapeDtypeStruct((_NP, 1), jnp.float32),
              jax.ShapeDtypeStruct((_NP, 1), jnp.float32)),
    scratch_types=[
        pltpu.VMEM((_K,), jnp.int32),
        pltpu.VMEM((_K, 1), jnp.float32),
        pltpu.VMEM_SHARED((_NP, 1), jnp.float32),
    ],
)
def _deg_kernel(dst_hbm, ones_hbm, zeros_hbm, out0, out1, didx, ones_v, acc):
    c = lax.axis_index("c")
    s = lax.axis_index("s")
    wid = s * _NC + c
    r0 = s * _TPR
    pltpu.sync_copy(zeros_hbm.at[pl.ds(r0, _TPR)], acc.at[pl.ds(r0, _TPR)])
    pltpu.sync_copy(ones_hbm, ones_v)
    plsc.subcore_barrier()
    base = wid * _EPW

    @pl.loop(0, _NCHUNK)
    def _(i):
        pltpu.sync_copy(dst_hbm.at[pl.ds(base + i * _K, _K)], didx)
        pltpu.sync_copy(ones_v, acc.at[didx], add=True)

    plsc.subcore_barrier()

    @pl.when(c == 0)
    def _():
        pltpu.sync_copy(acc.at[pl.ds(r0, _TPR)], out0.at[pl.ds(r0, _TPR)])

    @pl.when(c == 1)
    def _():
        pltpu.sync_copy(acc.at[pl.ds(r0, _TPR)], out1.at[pl.ds(r0, _TPR)])


def _make_agg(d):
    @functools.partial(
        pl.kernel,
        mesh=_mesh,
        compiler_params=_sc_params,
        out_type=(jax.ShapeDtypeStruct((_NP, d), jnp.float32),
                  jax.ShapeDtypeStruct((_NP, d), jnp.float32)),
        scratch_types=[
            pltpu.VMEM((_K,), jnp.int32),
            pltpu.VMEM((_K,), jnp.int32),
            pltpu.VMEM((_K, d), jnp.float32),
            pltpu.VMEM_SHARED((_NP, d), jnp.float32),
            pltpu.SemaphoreType.DMA,
        ],
    )
    def agg_kernel(table_hbm, src_hbm, dst_hbm, zeros_hbm, out0, out1,
                   sidx, didx, rows, acc, sem):
        c = lax.axis_index("c")
        s = lax.axis_index("s")
        wid = s * _NC + c
        r0 = s * _TPR
        pltpu.sync_copy(zeros_hbm.at[pl.ds(r0, _TPR)], acc.at[pl.ds(r0, _TPR)])
        plsc.subcore_barrier()
        base = wid * _EPW

        @pl.loop(0, _NCHUNK)
        def _(i):
            off = base + i * _K
            pltpu.sync_copy(src_hbm.at[pl.ds(off, _K)], sidx)
            pltpu.sync_copy(dst_hbm.at[pl.ds(off, _K)], didx)
            pltpu.async_copy(table_hbm.at[sidx], rows, sem).wait()
            pltpu.sync_copy(rows, acc.at[didx], add=True)

        plsc.subcore_barrier()

        @pl.when(c == 0)
        def _():
            pltpu.sync_copy(acc.at[pl.ds(r0, _TPR)], out0.at[pl.ds(r0, _TPR)])

        @pl.when(c == 1)
        def _():
            pltpu.sync_copy(acc.at[pl.ds(r0, _TPR)], out1.at[pl.ds(r0, _TPR)])

    return agg_kernel


_agg8 = _make_agg(_H)
_agg2 = _make_agg(_D2)


def _b1_body(x_ref, w1_ref, h_ref):
    h_ref[...] = jnp.dot(x_ref[...], w1_ref[...],
                         preferred_element_type=jnp.float32)


_b1 = pl.pallas_call(
    _b1_body,
    grid=(_GT,),
    in_specs=[pl.BlockSpec((_BT, _F_IN), lambda i: (i, 0)),
              pl.BlockSpec((_F_IN, _H), lambda i: (0, 0))],
    out_specs=pl.BlockSpec((_BT, _H), lambda i: (i, 0)),
    out_shape=jax.ShapeDtypeStruct((_N, _H), jnp.float32),
)


def _b2_body(c0_ref, c1_ref, h_ref, p1_ref, dinv_ref):
    deg = c0_ref[...] + c1_ref[...] + 1.0
    dv = lax.rsqrt(deg)
    dinv_ref[...] = dv
    p1_ref[...] = h_ref[...] * dv


_b2 = pl.pallas_call(
    _b2_body,
    grid=(_GT,),
    in_specs=[pl.BlockSpec((_BT, 1), lambda i: (i, 0)),
              pl.BlockSpec((_BT, 1), lambda i: (i, 0)),
              pl.BlockSpec((_BT, _H), lambda i: (i, 0))],
    out_specs=[pl.BlockSpec((_BT, _H), lambda i: (i, 0)),
               pl.BlockSpec((_BT, 1), lambda i: (i, 0))],
    out_shape=[jax.ShapeDtypeStruct((_N, _H), jnp.float32),
               jax.ShapeDtypeStruct((_N, 1), jnp.float32)],
)


def _d_body(a0_ref, a1_ref, p1_ref, dinv_ref, w2_ref, b1_ref, p2_ref):
    sagg = a0_ref[...] + a1_ref[...] + p1_ref[...]
    h1 = jnp.tanh(dinv_ref[...] * sagg + b1_ref[...])
    p2_ref[...] = jnp.dot(h1, w2_ref[...],
                          preferred_element_type=jnp.float32) * dinv_ref[...]


_d = pl.pallas_call(
    _d_body,
    grid=(_GT,),
    in_specs=[pl.BlockSpec((_BT, _H), lambda i: (i, 0)),
              pl.BlockSpec((_BT, _H), lambda i: (i, 0)),
              pl.BlockSpec((_BT, _H), lambda i: (i, 0)),
              pl.BlockSpec((_BT, 1), lambda i: (i, 0)),
              pl.BlockSpec((_H, _D2), lambda i: (0, 0)),
              pl.BlockSpec((1, _H), lambda i: (0, 0))],
    out_specs=pl.BlockSpec((_BT, _D2), lambda i: (i, 0)),
    out_shape=jax.ShapeDtypeStruct((_N, _D2), jnp.float32),
)


def _e_body(a0_ref, a1_ref, p2_ref, dinv_ref, wc_ref, b2_ref, bc_ref,
            out_ref, h2_ref):
    sagg = a0_ref[...] + a1_ref[...] + p2_ref[...]
    h2 = jnp.tanh(dinv_ref[...] * sagg + b2_ref[...])
    h2_ref[...] = h2
    out_ref[...] = jnp.dot(h2, wc_ref[...],
                           preferred_element_type=jnp.float32) + bc_ref[...]


_e = pl.pallas_call(
    _e_body,
    grid=(_GT,),
    in_specs=[pl.BlockSpec((_BT, _D2), lambda i: (i, 0)),
              pl.BlockSpec((_BT, _D2), lambda i: (i, 0)),
              pl.BlockSpec((_BT, _D2), lambda i: (i, 0)),
              pl.BlockSpec((_BT, 1), lambda i: (i, 0)),
              pl.BlockSpec((_D2, _C), lambda i: (0, 0)),
              pl.BlockSpec((1, _D2), lambda i: (0, 0)),
              pl.BlockSpec((1, _C), lambda i: (0, 0))],
    out_specs=[pl.BlockSpec((_BT, _C), lambda i: (i, 0)),
               pl.BlockSpec((_BT, _D2), lambda i: (i, 0))],
    out_shape=[jax.ShapeDtypeStruct((_N, _C), jnp.float32),
               jax.ShapeDtypeStruct((_N, _D2), jnp.float32)],
)


def kernel(x, edge_index, W1, b1, W2, b2, Wc, bc):
    src = edge_index[0]
    dst = edge_index[1]
    ones_k = jnp.ones((_K, 1), jnp.float32)
    z1 = jnp.zeros((_NP, 1), jnp.float32)
    z8 = jnp.zeros((_NP, _H), jnp.float32)
    z2 = jnp.zeros((_NP, _D2), jnp.float32)

    cnt0, cnt1 = _deg_kernel(dst, ones_k, z1)
    h = _b1(x, W1)
    p1, dinv = _b2(cnt0, cnt1, h)
    a0, a1 = _agg8(p1, src, dst, z8)
    p2 = _d(a0, a1, p1, dinv, W2, b1.reshape(1, _H))
    c0, c1 = _agg2(p2, src, dst, z2)
    out, h2 = _e(c0, c1, p2, dinv, Wc, b2.reshape(1, _D2), bc.reshape(1, _C))
    return (out, h2)
